# initial kernel scaffold (unmeasured)
import jax
import jax.numpy as jnp
from jax import lax
from jax.experimental import pallas as pl
from jax.experimental.pallas import tpu as pltpu

N_DEV = 16
SQ = 1024
SKV_PER = 1024
HQ = 8
DH = 128
D = HQ * DH
BLK = 64
SCALE = 0.08838834764831843
NEG = -1e9


def kernel(x, Wq, K_ext, V_ext, Wo):
    def body(x_ref, wq_ref, k_ref, v_ref, wo_ref, out_ref,
             comm_ctx, comm_st, ctx_send, ctx_recv, st_send, st_recv):
        my = lax.axis_index("i")
        left = (my - 1) % N_DEV
        right = (my + 1) % N_DEV

        barrier = pltpu.get_barrier_semaphore()
        for nbr in (left, right):
            pl.semaphore_signal(barrier, inc=1, device_id=(nbr,),
                                device_id_type=pl.DeviceIdType.MESH)
        pl.semaphore_wait(barrier, 2)

        q2d = jnp.dot(x_ref[0], wq_ref[...],
                      preferred_element_type=jnp.float32)

        qb = lax.broadcasted_iota(jnp.int32, (SQ, SKV_PER), 0) // BLK
        kb = (lax.broadcasted_iota(jnp.int32, (SQ, SKV_PER), 1) // BLK
              + my * (SKV_PER // BLK))
        mask = (qb == kb) | (kb == 0) | ((qb + kb) % 3 == 0)

        ctx_cols, m_cols, l_cols = [], [], []
        for h in range(HQ):
            q_h = q2d[:, h * DH:(h + 1) * DH]
            k_h = k_ref[0, :, h, :]
            v_h = v_ref[0, :, h, :]
            s = lax.dot_general(q_h, k_h, (((1,), (1,)), ((), ())),
                                preferred_element_type=jnp.float32) * SCALE
            s = jnp.where(mask, s, NEG)
            m_h = jnp.max(s, axis=1, keepdims=True)
            w = jnp.exp(s - m_h)
            l_h = jnp.sum(w, axis=1, keepdims=True)
            ctx_cols.append(jnp.dot(w, v_h,
                                    preferred_element_type=jnp.float32))
            m_cols.append(m_h)
            l_cols.append(l_h)

        ctx_a = jnp.concatenate(ctx_cols, axis=1)
        m_a = jnp.concatenate(m_cols, axis=1)
        l_a = jnp.concatenate(l_cols, axis=1)

        comm_ctx[0] = ctx_a
        comm_st[0] = jnp.concatenate([m_a, l_a], axis=1)

        for hop in range(N_DEV - 1):
            s_slot, r_slot = hop % 2, (hop + 1) % 2
            r1 = pltpu.make_async_remote_copy(
                src_ref=comm_ctx.at[s_slot], dst_ref=comm_ctx.at[r_slot],
                send_sem=ctx_send.at[s_slot], recv_sem=ctx_recv.at[r_slot],
                device_id=(right,), device_id_type=pl.DeviceIdType.MESH)
            r2 = pltpu.make_async_remote_copy(
                src_ref=comm_st.at[s_slot], dst_ref=comm_st.at[r_slot],
                send_sem=st_send.at[s_slot], recv_sem=st_recv.at[r_slot],
                device_id=(right,), device_id_type=pl.DeviceIdType.MESH)
            r1.start()
            r2.start()
            r1.wait()
            r2.wait()

            ctx_r = comm_ctx[r_slot]
            st_r = comm_st[r_slot]
            m_r = st_r[:, :HQ]
            l_r = st_r[:, HQ:]
            m_n = jnp.maximum(m_a, m_r)
            ea = jnp.exp(m_a - m_n)
            er = jnp.exp(m_r - m_n)
            l_a = l_a * ea + l_r * er
            ctx_a = jnp.concatenate(
                [ctx_a[:, h * DH:(h + 1) * DH] * ea[:, h:h + 1]
                 + ctx_r[:, h * DH:(h + 1) * DH] * er[:, h:h + 1]
                 for h in range(HQ)], axis=1)
            m_a = m_n

        inv_l = 1.0 / l_a
        ctx_f = jnp.concatenate(
            [ctx_a[:, h * DH:(h + 1) * DH] * inv_l[:, h:h + 1]
             for h in range(HQ)], axis=1)
        out_ref[0] = jnp.dot(ctx_f, wo_ref[...],
                             preferred_element_type=jnp.float32)

    return pl.pallas_call(
        body,
        out_shape=jax.ShapeDtypeStruct((1, SQ, D), jnp.float32),
        in_specs=[pl.BlockSpec(memory_space=pltpu.VMEM)] * 5,
        out_specs=pl.BlockSpec(memory_space=pltpu.VMEM),
        scratch_shapes=[
            pltpu.VMEM((2, SQ, D), jnp.float32),
            pltpu.VMEM((2, SQ, 2 * HQ), jnp.float32),
            pltpu.SemaphoreType.DMA((2,)),
            pltpu.SemaphoreType.DMA((2,)),
            pltpu.SemaphoreType.DMA((2,)),
            pltpu.SemaphoreType.DMA((2,)),
        ],
        compiler_params=pltpu.CompilerParams(collective_id=0),
    )(x, Wq, K_ext, V_ext, Wo)


# baseline (device time: 895566 ns/iter reference)
import jax
import jax.numpy as jnp
from jax import lax
from jax.experimental import pallas as pl
from jax.experimental.pallas import tpu as pltpu

N_DEV = 16
SQ = 1024
SKV_PER = 1024
HQ = 8
DH = 128
D = HQ * DH
BLK = 64
SCALE = 0.08838834764831843
NEG = -1e9


QC = 256


def _attn_body(x_ref, wq_ref, k_ref, v_ref, ctx_ref, m_ref, l_ref):
    my = lax.axis_index("i")
    qc0 = pl.program_id(0) * QC
    q_chunk = jnp.dot(x_ref[0], wq_ref[...],
                      preferred_element_type=jnp.float32)
    qb = (lax.broadcasted_iota(jnp.int32, (QC, SKV_PER), 0) + qc0) // BLK
    kb = (lax.broadcasted_iota(jnp.int32, (QC, SKV_PER), 1) // BLK
          + my * (SKV_PER // BLK))
    mask = (qb == kb) | (kb == 0) | ((qb + kb) % 3 == 0)
    for h in range(HQ):
        q_h = q_chunk[:, h * DH:(h + 1) * DH]
        k_h = k_ref[0, :, h, :]
        v_h = v_ref[0, :, h, :]
        s = lax.dot_general(q_h, k_h, (((1,), (1,)), ((), ())),
                            preferred_element_type=jnp.float32) * SCALE
        s = jnp.where(mask, s, NEG)
        m_h = jnp.max(s, axis=1, keepdims=True)
        w = jnp.exp(s - m_h)
        l_h = jnp.sum(w, axis=1, keepdims=True)
        ctx_ref[:, h, :] = jnp.dot(w, v_h,
                                   preferred_element_type=jnp.float32)
        m_ref[:, h:h + 1] = m_h
        l_ref[:, h:h + 1] = l_h


def _ring_body(ctx_ref, mm_ref, ll_ref, wo_ref, out_ref,
               comm_ctx, comm_st, acc_ctx, acc_st,
               ctx_send, ctx_recv, st_send, st_recv):
    my = lax.axis_index("i")
    left = (my - 1) % N_DEV
    right = (my + 1) % N_DEV

    barrier = pltpu.get_barrier_semaphore()
    for nbr in (left, right):
        pl.semaphore_signal(barrier, inc=1, device_id=(nbr,),
                            device_id_type=pl.DeviceIdType.MESH)
    pl.semaphore_wait(barrier, 2)

    acc_ctx[...] = ctx_ref[...]
    acc_st[:, :HQ] = mm_ref[...]
    acc_st[:, HQ:] = ll_ref[...]
    comm_ctx[0] = ctx_ref[...]
    comm_st[0, :, :HQ] = mm_ref[...]
    comm_st[0, :, HQ:] = ll_ref[...]

    def hop_body(hop, carry):
        s_slot = lax.rem(hop, 2)
        r_slot = 1 - s_slot
        r1 = pltpu.make_async_remote_copy(
            src_ref=comm_ctx.at[s_slot], dst_ref=comm_ctx.at[r_slot],
            send_sem=ctx_send.at[s_slot], recv_sem=ctx_recv.at[r_slot],
            device_id=(right,), device_id_type=pl.DeviceIdType.MESH)
        r2 = pltpu.make_async_remote_copy(
            src_ref=comm_st.at[s_slot], dst_ref=comm_st.at[r_slot],
            send_sem=st_send.at[s_slot], recv_sem=st_recv.at[r_slot],
            device_id=(right,), device_id_type=pl.DeviceIdType.MESH)
        r1.start()
        r2.start()
        r1.wait()
        r2.wait()

        m_a = acc_st[:, :HQ]
        l_a = acc_st[:, HQ:]
        m_r = comm_st[r_slot, :, :HQ]
        l_r = comm_st[r_slot, :, HQ:]
        m_n = jnp.maximum(m_a, m_r)
        ea = jnp.exp(m_a - m_n)
        er = jnp.exp(m_r - m_n)
        acc_st[:, :HQ] = m_n
        acc_st[:, HQ:] = l_a * ea + l_r * er
        acc_ctx[...] = (acc_ctx[...] * ea[:, :, None]
                        + comm_ctx[r_slot] * er[:, :, None])
        return carry

    lax.fori_loop(0, N_DEV - 1, hop_body, 0)

    inv_l = 1.0 / acc_st[:, HQ:]
    acc_ctx[...] = acc_ctx[...] * inv_l[:, :, None]

    def proj_body(i, carry):
        qc = i * 256
        o = jnp.zeros((256, D), dtype=jnp.float32)
        for h in range(HQ):
            o = o + jnp.dot(acc_ctx[pl.ds(qc, 256), h],
                            wo_ref[h * DH:(h + 1) * DH, :],
                            preferred_element_type=jnp.float32)
        out_ref[0, pl.ds(qc, 256), :] = o
        return carry

    lax.fori_loop(0, SQ // 256, proj_body, 0)


def kernel(x, Wq, K_ext, V_ext, Wo):
    ctx, mm, ll = pl.pallas_call(
        _attn_body,
        grid=(SQ // QC,),
        out_shape=[
            jax.ShapeDtypeStruct((SQ, HQ, DH), jnp.float32),
            jax.ShapeDtypeStruct((SQ, HQ), jnp.float32),
            jax.ShapeDtypeStruct((SQ, HQ), jnp.float32),
        ],
        in_specs=[
            pl.BlockSpec((1, QC, D), lambda i: (0, i, 0)),
            pl.BlockSpec((D, D), lambda i: (0, 0)),
            pl.BlockSpec((1, SKV_PER, HQ, DH), lambda i: (0, 0, 0, 0)),
            pl.BlockSpec((1, SKV_PER, HQ, DH), lambda i: (0, 0, 0, 0)),
        ],
        out_specs=[
            pl.BlockSpec((QC, HQ, DH), lambda i: (i, 0, 0)),
            pl.BlockSpec((QC, HQ), lambda i: (i, 0)),
            pl.BlockSpec((QC, HQ), lambda i: (i, 0)),
        ],
        compiler_params=pltpu.CompilerParams(
            vmem_limit_bytes=63 * 1024 * 1024),
    )(x, Wq, K_ext, V_ext)

    return pl.pallas_call(
        _ring_body,
        out_shape=jax.ShapeDtypeStruct((1, SQ, D), jnp.float32),
        in_specs=[pl.BlockSpec(memory_space=pltpu.VMEM)] * 4,
        out_specs=pl.BlockSpec(memory_space=pltpu.VMEM),
        scratch_shapes=[
            pltpu.VMEM((2, SQ, HQ, DH), jnp.float32),
            pltpu.VMEM((2, SQ, 2 * HQ), jnp.float32),
            pltpu.VMEM((SQ, HQ, DH), jnp.float32),
            pltpu.VMEM((SQ, 2 * HQ), jnp.float32),
            pltpu.SemaphoreType.DMA((2,)),
            pltpu.SemaphoreType.DMA((2,)),
            pltpu.SemaphoreType.DMA((2,)),
            pltpu.SemaphoreType.DMA((2,)),
        ],
        compiler_params=pltpu.CompilerParams(
            collective_id=0, vmem_limit_bytes=63 * 1024 * 1024),
    )(ctx, mm, ll, Wo)


# device time: 167052 ns/iter; 5.3610x vs baseline; 5.3610x over previous
import jax
import jax.numpy as jnp
from jax import lax
from jax.experimental import pallas as pl
from jax.experimental.pallas import tpu as pltpu

N_DEV = 16
SQ = 1024
SKV_PER = 1024
HQ = 8
DH = 128
D = HQ * DH
BLK = 64
CH = SQ // N_DEV
QC = 256
SCALE = 0.08838834764831843
NEG = -1e9

_MESH = pl.DeviceIdType.MESH


def _attn_body(x_ref, wq_ref, k_ref, v_ref, ctx_ref, st_ref):
    my = lax.axis_index("i")
    qc0 = pl.program_id(0) * QC
    q_chunk = jnp.dot(x_ref[0], wq_ref[...],
                      preferred_element_type=jnp.float32)
    qb = (lax.broadcasted_iota(jnp.int32, (QC, SKV_PER), 0) + qc0) // BLK
    kb = (lax.broadcasted_iota(jnp.int32, (QC, SKV_PER), 1) // BLK
          + my * (SKV_PER // BLK))
    mask = (qb == kb) | (kb == 0) | ((qb + kb) % 3 == 0)
    for h in range(HQ):
        q_h = q_chunk[:, h * DH:(h + 1) * DH]
        k_h = k_ref[0, :, h, :]
        v_h = v_ref[0, :, h, :]
        s = lax.dot_general(q_h, k_h, (((1,), (1,)), ((), ())),
                            preferred_element_type=jnp.float32) * SCALE
        s = jnp.where(mask, s, NEG)
        m_h = jnp.max(s, axis=1, keepdims=True)
        w = jnp.exp(s - m_h)
        l_h = jnp.sum(w, axis=1, keepdims=True)
        ctx_h = jnp.dot(w, v_h, preferred_element_type=jnp.float32)
        for c in range(QC // CH):
            rows = slice(c * CH, (c + 1) * CH)
            ctx_ref[c, :, h, :] = ctx_h[rows, :]
            st_ref[c, :, h:h + 1] = m_h[rows, :]
            st_ref[c, :, HQ + h:HQ + h + 1] = l_h[rows, :]


def _combine(ctx_a, st_a, ctx_r, st_r):
    m_a, l_a = st_a[:, :HQ], st_a[:, HQ:]
    m_r, l_r = st_r[:, :HQ], st_r[:, HQ:]
    m_n = jnp.maximum(m_a, m_r)
    ea = jnp.exp(m_a - m_n)
    er = jnp.exp(m_r - m_n)
    ctx_n = ctx_a * ea[:, :, None] + ctx_r * er[:, :, None]
    st_n = jnp.concatenate([m_n, l_a * ea + l_r * er], axis=1)
    return ctx_n, st_n


def _ring_body(ctx_ref, st_ref, wo_ref, out_ref,
               rs_ctx, rs_st, bc_buf,
               rs_ctx_send, rs_ctx_recv, rs_st_send, rs_st_recv,
               bc_send, bc_recv):
    my = lax.axis_index("i")

    barrier = pltpu.get_barrier_semaphore()

    def _peer(j):
        return j + jnp.where(j >= my, 1, 0)

    def bar_sig(j, c):
        pl.semaphore_signal(barrier, inc=1, device_id=(_peer(j),),
                            device_id_type=_MESH)
        return c
    lax.fori_loop(0, N_DEV - 1, bar_sig, 0)
    pl.semaphore_wait(barrier, N_DEV - 1)

    def rs_send(j, c):
        jj = _peer(j)
        r1 = pltpu.make_async_remote_copy(
            src_ref=ctx_ref.at[jj], dst_ref=rs_ctx.at[my],
            send_sem=rs_ctx_send.at[jj], recv_sem=rs_ctx_recv.at[my],
            device_id=(jj,), device_id_type=_MESH)
        r2 = pltpu.make_async_remote_copy(
            src_ref=st_ref.at[jj], dst_ref=rs_st.at[my],
            send_sem=rs_st_send.at[jj], recv_sem=rs_st_recv.at[my],
            device_id=(jj,), device_id_type=_MESH)
        r1.start()
        r2.start()
        return c
    lax.fori_loop(0, N_DEV - 1, rs_send, 0)

    def rs_recv(j, carry):
        ctx_a, st_a = carry
        jj = _peer(j)
        d1 = pltpu.make_async_remote_copy(
            src_ref=rs_ctx.at[jj], dst_ref=rs_ctx.at[jj],
            send_sem=rs_ctx_send.at[jj], recv_sem=rs_ctx_recv.at[jj],
            device_id=(my,), device_id_type=_MESH)
        d2 = pltpu.make_async_remote_copy(
            src_ref=rs_st.at[jj], dst_ref=rs_st.at[jj],
            send_sem=rs_st_send.at[jj], recv_sem=rs_st_recv.at[jj],
            device_id=(my,), device_id_type=_MESH)
        d1.wait_recv()
        d2.wait_recv()
        return _combine(ctx_a, st_a, rs_ctx[jj], rs_st[jj])

    ctx_f, st_f = lax.fori_loop(
        0, N_DEV - 1, rs_recv, (ctx_ref[my], st_ref[my]))

    inv_l = 1.0 / st_f[:, HQ:]
    ctx_n = ctx_f * inv_l[:, :, None]
    o = jnp.zeros((CH, D), dtype=jnp.float32)
    for h in range(HQ):
        o = o + jnp.dot(ctx_n[:, h, :], wo_ref[h * DH:(h + 1) * DH, :],
                        preferred_element_type=jnp.float32)
    out_ref[0, pl.ds(my * CH, CH), :] = o
    bc_buf[my] = o

    def bc_push(j, c):
        jj = _peer(j)
        r = pltpu.make_async_remote_copy(
            src_ref=bc_buf.at[my], dst_ref=bc_buf.at[my],
            send_sem=bc_send.at[jj], recv_sem=bc_recv.at[my],
            device_id=(jj,), device_id_type=_MESH)
        r.start()
        return c
    lax.fori_loop(0, N_DEV - 1, bc_push, 0)

    def bc_pull(j, c):
        jj = _peer(j)
        d = pltpu.make_async_remote_copy(
            src_ref=bc_buf.at[jj], dst_ref=bc_buf.at[jj],
            send_sem=bc_send.at[jj], recv_sem=bc_recv.at[jj],
            device_id=(my,), device_id_type=_MESH)
        d.wait_recv()
        out_ref[0, pl.ds(jj * CH, CH), :] = bc_buf[jj]
        return c
    lax.fori_loop(0, N_DEV - 1, bc_pull, 0)

    def drain(j, c):
        jj = _peer(j)
        d1 = pltpu.make_async_remote_copy(
            src_ref=ctx_ref.at[jj], dst_ref=rs_ctx.at[jj],
            send_sem=rs_ctx_send.at[jj], recv_sem=rs_ctx_recv.at[jj],
            device_id=(my,), device_id_type=_MESH)
        d2 = pltpu.make_async_remote_copy(
            src_ref=st_ref.at[jj], dst_ref=rs_st.at[jj],
            send_sem=rs_st_send.at[jj], recv_sem=rs_st_recv.at[jj],
            device_id=(my,), device_id_type=_MESH)
        d3 = pltpu.make_async_remote_copy(
            src_ref=bc_buf.at[my], dst_ref=bc_buf.at[my],
            send_sem=bc_send.at[jj], recv_sem=bc_recv.at[jj],
            device_id=(my,), device_id_type=_MESH)
        d1.wait_send()
        d2.wait_send()
        d3.wait_send()
        return c
    lax.fori_loop(0, N_DEV - 1, drain, 0)


def kernel(x, Wq, K_ext, V_ext, Wo):
    ctx, st = pl.pallas_call(
        _attn_body,
        grid=(SQ // QC,),
        out_shape=[
            jax.ShapeDtypeStruct((N_DEV, CH, HQ, DH), jnp.float32),
            jax.ShapeDtypeStruct((N_DEV, CH, 2 * HQ), jnp.float32),
        ],
        in_specs=[
            pl.BlockSpec((1, QC, D), lambda i: (0, i, 0)),
            pl.BlockSpec((D, D), lambda i: (0, 0)),
            pl.BlockSpec((1, SKV_PER, HQ, DH), lambda i: (0, 0, 0, 0)),
            pl.BlockSpec((1, SKV_PER, HQ, DH), lambda i: (0, 0, 0, 0)),
        ],
        out_specs=[
            pl.BlockSpec((QC // CH, CH, HQ, DH), lambda i: (i, 0, 0, 0)),
            pl.BlockSpec((QC // CH, CH, 2 * HQ), lambda i: (i, 0, 0)),
        ],
        compiler_params=pltpu.CompilerParams(
            vmem_limit_bytes=63 * 1024 * 1024),
    )(x, Wq, K_ext, V_ext)

    return pl.pallas_call(
        _ring_body,
        out_shape=jax.ShapeDtypeStruct((1, SQ, D), jnp.float32),
        in_specs=[pl.BlockSpec(memory_space=pltpu.VMEM)] * 3,
        out_specs=pl.BlockSpec(memory_space=pltpu.VMEM),
        scratch_shapes=[
            pltpu.VMEM((N_DEV, CH, HQ, DH), jnp.float32),
            pltpu.VMEM((N_DEV, CH, 2 * HQ), jnp.float32),
            pltpu.VMEM((N_DEV, CH, D), jnp.float32),
            pltpu.SemaphoreType.DMA((N_DEV,)),
            pltpu.SemaphoreType.DMA((N_DEV,)),
            pltpu.SemaphoreType.DMA((N_DEV,)),
            pltpu.SemaphoreType.DMA((N_DEV,)),
            pltpu.SemaphoreType.DMA((N_DEV,)),
            pltpu.SemaphoreType.DMA((N_DEV,)),
        ],
        compiler_params=pltpu.CompilerParams(
            collective_id=0, vmem_limit_bytes=63 * 1024 * 1024),
    )(ctx, st, Wo)
